# Initial kernel scaffold; baseline (speedup 1.0000x reference)
import jax
import jax.numpy as jnp
from jax.experimental import pallas as pl


def kernel(x, edge_index, W1, b1, W2, b2, W3, b3):
    raise NotImplementedError("write your pallas kernel here")


# speedup vs baseline: 16.8754x; 16.8754x over previous
import functools

import jax
import jax.numpy as jnp
from jax import lax
from jax.experimental import pallas as pl
from jax.experimental.pallas import tpu as pltpu
from jax.experimental.pallas import tpu_sc as plsc

N = 10000
D = 128
E = 320000
NC, NS = 2, 16
NW = NC * NS
EPT = E // NW
CH = 80
NCHUNK = EPT // CH
ST = 624
STL = N - (NS - 1) * ST

_sc_mesh = plsc.VectorSubcoreMesh(
    core_axis_name="c", subcore_axis_name="s", num_cores=NC, num_subcores=NS
)


@functools.partial(
    pl.kernel,
    out_type=jax.ShapeDtypeStruct((NC, N), jnp.float32),
    mesh=_sc_mesh,
    scratch_types=[
        pltpu.VMEM_SHARED((N,), jnp.float32),
        pltpu.VMEM((2, CH), jnp.int32),
        pltpu.VMEM((1, CH), jnp.float32),
    ],
)
def _sc_hist(dst_hbm, zeros1_hbm, out_hbm, acc, idx_v, ones_v):
    c = lax.axis_index("c")
    s = lax.axis_index("s")
    wid = s * NC + c
    base = wid * EPT
    for j in range(CH // 16):
        ones_v[0, pl.ds(j * 16, 16)] = jnp.full((16,), 1.0, jnp.float32)

    @pl.when(s == 0)
    def _():
        pltpu.sync_copy(zeros1_hbm, acc)

    plsc.subcore_barrier()

    def body(i, carry):
        pltpu.sync_copy(dst_hbm.at[pl.ds(base + i * CH, CH)], idx_v.at[0])
        pltpu.sync_copy(ones_v.at[0], acc.at[idx_v.at[0]], add=True)
        return carry

    lax.fori_loop(0, NCHUNK, body, 0)
    plsc.subcore_barrier()

    @pl.when(s == 0)
    def _():
        pltpu.sync_copy(acc, out_hbm.at[c])


@functools.partial(
    pl.kernel,
    out_type=jax.ShapeDtypeStruct((NC, N, D), jnp.float32),
    mesh=_sc_mesh,
    scratch_types=[
        pltpu.VMEM_SHARED((N, D), jnp.float32),
        pltpu.VMEM((2, CH), jnp.int32),
        pltpu.VMEM((2, CH), jnp.int32),
        pltpu.VMEM((2, CH, D), jnp.float32),
        pltpu.SemaphoreType.DMA,
        pltpu.SemaphoreType.DMA,
    ],
)
def _sc_scatter(y_hbm, src_hbm, dst_hbm, zeros2_hbm, out_hbm,
                acc, src_v, dst_v, rows_v, sem0, sem1):
    c = lax.axis_index("c")
    s = lax.axis_index("s")
    wid = s * NC + c
    base = wid * EPT

    @pl.when(s < NS - 1)
    def _():
        off = pl.multiple_of(s * ST, 8)
        pltpu.sync_copy(zeros2_hbm.at[pl.ds(off, ST)], acc.at[pl.ds(off, ST)])

    @pl.when(s == NS - 1)
    def _():
        pltpu.sync_copy(zeros2_hbm.at[pl.ds((NS - 1) * ST, STL)],
                        acc.at[pl.ds((NS - 1) * ST, STL)])

    plsc.subcore_barrier()

    pltpu.sync_copy(src_hbm.at[pl.ds(base, CH)], src_v.at[0])
    pltpu.sync_copy(dst_hbm.at[pl.ds(base, CH)], dst_v.at[0])
    pltpu.async_copy(y_hbm.at[src_v.at[0]], rows_v.at[0], sem0)

    def body(i, carry):
        b = lax.rem(i, 2)

        @pl.when(i + 1 < NCHUNK)
        def _():
            nb = lax.rem(i + 1, 2)

            @pl.when(nb == 0)
            def _():
                pltpu.sync_copy(
                    src_hbm.at[pl.ds(base + (i + 1) * CH, CH)], src_v.at[0])
                pltpu.sync_copy(
                    dst_hbm.at[pl.ds(base + (i + 1) * CH, CH)], dst_v.at[0])
                pltpu.async_copy(y_hbm.at[src_v.at[0]], rows_v.at[0], sem0)

            @pl.when(nb == 1)
            def _():
                pltpu.sync_copy(
                    src_hbm.at[pl.ds(base + (i + 1) * CH, CH)], src_v.at[1])
                pltpu.sync_copy(
                    dst_hbm.at[pl.ds(base + (i + 1) * CH, CH)], dst_v.at[1])
                pltpu.async_copy(y_hbm.at[src_v.at[1]], rows_v.at[1], sem1)

        @pl.when(b == 0)
        def _():
            pltpu.make_async_copy(y_hbm.at[src_v.at[0]], rows_v.at[0],
                                  sem0).wait()
            pltpu.sync_copy(rows_v.at[0], acc.at[dst_v.at[0]], add=True)

        @pl.when(b == 1)
        def _():
            pltpu.make_async_copy(y_hbm.at[src_v.at[1]], rows_v.at[1],
                                  sem1).wait()
            pltpu.sync_copy(rows_v.at[1], acc.at[dst_v.at[1]], add=True)

        return carry

    lax.fori_loop(0, NCHUNK, body, 0)
    plsc.subcore_barrier()

    @pl.when(s < NS - 1)
    def _():
        off = pl.multiple_of(s * ST, 8)
        pltpu.sync_copy(acc.at[pl.ds(off, ST)],
                        out_hbm.at[c, pl.ds(off, ST)])

    @pl.when(s == NS - 1)
    def _():
        pltpu.sync_copy(acc.at[pl.ds((NS - 1) * ST, STL)],
                        out_hbm.at[c, pl.ds((NS - 1) * ST, STL)])


BR = 2000
_row_grid = N // BR


def _tc_prep_body(hist_ref, x_ref, w_ref, dinv_ref, y_ref):
    hist = hist_ref[0] + hist_ref[1]
    dinv = lax.rsqrt(hist + 1.0)
    xw = jnp.dot(x_ref[...], w_ref[...], preferred_element_type=jnp.float32)
    dinv_ref[...] = dinv
    y_ref[...] = xw * dinv


_tc_prep = pl.pallas_call(
    _tc_prep_body,
    grid=(_row_grid,),
    in_specs=[
        pl.BlockSpec((NC, BR, 1), lambda i: (0, i, 0)),
        pl.BlockSpec((BR, D), lambda i: (i, 0)),
        pl.BlockSpec((D, D), lambda i: (0, 0)),
    ],
    out_specs=[
        pl.BlockSpec((BR, 1), lambda i: (i, 0)),
        pl.BlockSpec((BR, D), lambda i: (i, 0)),
    ],
    out_shape=[
        jax.ShapeDtypeStruct((N, 1), jnp.float32),
        jax.ShapeDtypeStruct((N, D), jnp.float32),
    ],
)


def _tc_mid_body(p_ref, y_ref, dinv_ref, b_ref, w_ref, o_ref):
    z = dinv_ref[...] * (p_ref[0] + p_ref[1] + y_ref[...]) + b_ref[...]
    h = jnp.maximum(z, 0.0)
    hw = jnp.dot(h, w_ref[...], preferred_element_type=jnp.float32)
    o_ref[...] = hw * dinv_ref[...]


_tc_mid = pl.pallas_call(
    _tc_mid_body,
    grid=(_row_grid,),
    in_specs=[
        pl.BlockSpec((NC, BR, D), lambda i: (0, i, 0)),
        pl.BlockSpec((BR, D), lambda i: (i, 0)),
        pl.BlockSpec((BR, 1), lambda i: (i, 0)),
        pl.BlockSpec((1, D), lambda i: (0, 0)),
        pl.BlockSpec((D, D), lambda i: (0, 0)),
    ],
    out_specs=pl.BlockSpec((BR, D), lambda i: (i, 0)),
    out_shape=jax.ShapeDtypeStruct((N, D), jnp.float32),
)


def _tc_final_body(p_ref, y_ref, dinv_ref, b_ref, o_ref):
    z = dinv_ref[...] * (p_ref[0] + p_ref[1] + y_ref[...]) + b_ref[...]
    m = jnp.max(z, axis=1, keepdims=True)
    zs = z - m
    lse = jnp.log(jnp.sum(jnp.exp(zs), axis=1, keepdims=True))
    o_ref[...] = zs - lse


_tc_final = pl.pallas_call(
    _tc_final_body,
    grid=(_row_grid,),
    in_specs=[
        pl.BlockSpec((NC, BR, D), lambda i: (0, i, 0)),
        pl.BlockSpec((BR, D), lambda i: (i, 0)),
        pl.BlockSpec((BR, 1), lambda i: (i, 0)),
        pl.BlockSpec((1, D), lambda i: (0, 0)),
    ],
    out_specs=pl.BlockSpec((BR, D), lambda i: (i, 0)),
    out_shape=jax.ShapeDtypeStruct((N, D), jnp.float32),
)


def kernel(x, edge_index, W1, b1, W2, b2, W3, b3):
    src = edge_index[0].astype(jnp.int32)
    dst = edge_index[1].astype(jnp.int32)
    zeros1 = jnp.zeros((N,), jnp.float32)
    zeros2 = jnp.zeros((N, D), jnp.float32)

    hist_p = _sc_hist(dst, zeros1)
    hist_p = hist_p.reshape(NC, N, 1)
    dinv, y = _tc_prep(hist_p, x, W1)

    p = _sc_scatter(y, src, dst, zeros2)
    y = _tc_mid(p, y, dinv, b1.reshape(1, D), W2)

    p = _sc_scatter(y, src, dst, zeros2)
    y = _tc_mid(p, y, dinv, b2.reshape(1, D), W3)

    p = _sc_scatter(y, src, dst, zeros2)
    return _tc_final(p, y, dinv, b3.reshape(1, D))


# speedup vs baseline: 20.5363x; 1.2169x over previous
import functools

import jax
import jax.numpy as jnp
from jax import lax
from jax.experimental import pallas as pl
from jax.experimental.pallas import tpu as pltpu
from jax.experimental.pallas import tpu_sc as plsc

N = 10000
D = 128
DH = D // 2
E = 320000
NC, NS = 2, 16
NW = NC * NS
CH = 80

EPT_H = E // NW
NCH_H = EPT_H // CH
EPT_S = E // NS
NCH_S = EPT_S // CH

NBUF = 5
NG_H = NCH_H // NBUF
NG_S = NCH_S // NBUF

ST = 624
STL = N - (NS - 1) * ST

_sc_mesh = plsc.VectorSubcoreMesh(
    core_axis_name="c", subcore_axis_name="s", num_cores=NC, num_subcores=NS
)


@functools.partial(
    pl.kernel,
    out_type=jax.ShapeDtypeStruct((NC, N), jnp.float32),
    mesh=_sc_mesh,
    scratch_types=[
        pltpu.VMEM_SHARED((N,), jnp.float32),
        pltpu.VMEM((NCH_H, CH), jnp.int32),
        pltpu.VMEM((1, CH), jnp.float32),
        pltpu.SemaphoreType.DMA,
    ],
)
def _sc_hist(dst3_hbm, zeros1_hbm, out_hbm, acc, idx_v, ones_v, sem):
    c = lax.axis_index("c")
    s = lax.axis_index("s")
    wid = s * NC + c
    for j in range(CH // 16):
        ones_v[0, pl.ds(j * 16, 16)] = jnp.full((16,), 1.0, jnp.float32)

    @pl.when(s == 0)
    def _():
        pltpu.sync_copy(zeros1_hbm, acc)

    pltpu.sync_copy(dst3_hbm.at[wid], idx_v)
    plsc.subcore_barrier()

    def grp(g, carry):
        for b in range(NBUF):
            pltpu.async_copy(ones_v.at[0], acc.at[idx_v.at[g * NBUF + b]],
                             sem, add=True)
        for b in range(NBUF):
            pltpu.make_async_copy(ones_v.at[0],
                                  acc.at[idx_v.at[g * NBUF + b]], sem).wait()
        return carry

    lax.fori_loop(0, NG_H, grp, 0)
    plsc.subcore_barrier()

    @pl.when(s == 0)
    def _():
        pltpu.sync_copy(acc, out_hbm.at[c])


@functools.partial(
    pl.kernel,
    out_type=jax.ShapeDtypeStruct((NC, N, DH), jnp.float32),
    mesh=_sc_mesh,
    scratch_types=[
        pltpu.VMEM_SHARED((N, DH), jnp.float32),
        pltpu.VMEM((NCH_S, CH), jnp.int32),
        pltpu.VMEM((NCH_S, CH), jnp.int32),
        pltpu.VMEM((NBUF, CH, DH), jnp.float32),
        [pltpu.SemaphoreType.DMA] * NBUF,
        pltpu.SemaphoreType.DMA,
    ],
    compiler_params=pltpu.CompilerParams(use_tc_tiling_on_sc=False),
)
def _sc_scatter(y_hbm, src3_hbm, dst3_hbm, zeros2_hbm, out_hbm,
                acc, src_i, dst_i, rows, gsems, ssem):
    c = lax.axis_index("c")
    s = lax.axis_index("s")

    @pl.when(s < NS - 1)
    def _():
        off = pl.multiple_of(s * ST, 8)
        pltpu.sync_copy(zeros2_hbm.at[pl.ds(off, ST)], acc.at[pl.ds(off, ST)])

    @pl.when(s == NS - 1)
    def _():
        pltpu.sync_copy(zeros2_hbm.at[pl.ds((NS - 1) * ST, STL)],
                        acc.at[pl.ds((NS - 1) * ST, STL)])

    pltpu.sync_copy(src3_hbm.at[s], src_i)
    pltpu.sync_copy(dst3_hbm.at[s], dst_i)
    plsc.subcore_barrier()

    yh = y_hbm.at[c]

    def grp(g, carry):
        base_ch = g * NBUF

        @pl.when(g > 0)
        def _():
            for b in range(NBUF):
                pltpu.make_async_copy(
                    rows.at[b], acc.at[dst_i.at[base_ch - NBUF + b]],
                    ssem).wait()

        for b in range(NBUF):
            pltpu.async_copy(yh.at[src_i.at[base_ch + b]], rows.at[b],
                             gsems[b])
        for b in range(NBUF):
            pltpu.make_async_copy(yh.at[src_i.at[base_ch + b]],
                                  rows.at[b], gsems[b]).wait()
            pltpu.async_copy(rows.at[b], acc.at[dst_i.at[base_ch + b]],
                             ssem, add=True)
        return carry

    lax.fori_loop(0, NG_S, grp, 0)
    for b in range(NBUF):
        pltpu.make_async_copy(rows.at[b],
                              acc.at[dst_i.at[(NG_S - 1) * NBUF + b]],
                              ssem).wait()
    plsc.subcore_barrier()

    @pl.when(s < NS - 1)
    def _():
        off = pl.multiple_of(s * ST, 8)
        pltpu.sync_copy(acc.at[pl.ds(off, ST)],
                        out_hbm.at[c, pl.ds(off, ST)])

    @pl.when(s == NS - 1)
    def _():
        pltpu.sync_copy(acc.at[pl.ds((NS - 1) * ST, STL)],
                        out_hbm.at[c, pl.ds((NS - 1) * ST, STL)])


BR = 2000
_row_grid = N // BR


def _split(z):
    return jnp.stack([z[:, :DH], z[:, DH:]])


def _tc_prep_body(hist_ref, x_ref, w_ref, dinv_ref, y_ref):
    hist = hist_ref[0] + hist_ref[1]
    dinv = lax.rsqrt(hist + 1.0)
    xw = jnp.dot(x_ref[...], w_ref[...], preferred_element_type=jnp.float32)
    dinv_ref[...] = dinv
    y_ref[...] = _split(xw * dinv)


_tc_prep = pl.pallas_call(
    _tc_prep_body,
    grid=(_row_grid,),
    in_specs=[
        pl.BlockSpec((NC, BR, 1), lambda i: (0, i, 0)),
        pl.BlockSpec((BR, D), lambda i: (i, 0)),
        pl.BlockSpec((D, D), lambda i: (0, 0)),
    ],
    out_specs=[
        pl.BlockSpec((BR, 1), lambda i: (i, 0)),
        pl.BlockSpec((NC, BR, DH), lambda i: (0, i, 0)),
    ],
    out_shape=[
        jax.ShapeDtypeStruct((N, 1), jnp.float32),
        jax.ShapeDtypeStruct((NC, N, DH), jnp.float32),
    ],
)


def _tc_mid_body(p_ref, y_ref, dinv_ref, b_ref, w_ref, o_ref):
    dinv = dinv_ref[...]
    z = jnp.concatenate(
        [dinv * (p_ref[0] + y_ref[0]), dinv * (p_ref[1] + y_ref[1])], axis=1)
    h = jnp.maximum(z + b_ref[...], 0.0)
    hw = jnp.dot(h, w_ref[...], preferred_element_type=jnp.float32)
    o_ref[...] = _split(hw * dinv)


_tc_mid = pl.pallas_call(
    _tc_mid_body,
    grid=(_row_grid,),
    in_specs=[
        pl.BlockSpec((NC, BR, DH), lambda i: (0, i, 0)),
        pl.BlockSpec((NC, BR, DH), lambda i: (0, i, 0)),
        pl.BlockSpec((BR, 1), lambda i: (i, 0)),
        pl.BlockSpec((1, D), lambda i: (0, 0)),
        pl.BlockSpec((D, D), lambda i: (0, 0)),
    ],
    out_specs=pl.BlockSpec((NC, BR, DH), lambda i: (0, i, 0)),
    out_shape=jax.ShapeDtypeStruct((NC, N, DH), jnp.float32),
)


def _tc_final_body(p_ref, y_ref, dinv_ref, b_ref, o_ref):
    dinv = dinv_ref[...]
    z = jnp.concatenate(
        [dinv * (p_ref[0] + y_ref[0]), dinv * (p_ref[1] + y_ref[1])], axis=1)
    z = z + b_ref[...]
    m = jnp.max(z, axis=1, keepdims=True)
    zs = z - m
    lse = jnp.log(jnp.sum(jnp.exp(zs), axis=1, keepdims=True))
    o_ref[...] = zs - lse


_tc_final = pl.pallas_call(
    _tc_final_body,
    grid=(_row_grid,),
    in_specs=[
        pl.BlockSpec((NC, BR, DH), lambda i: (0, i, 0)),
        pl.BlockSpec((NC, BR, DH), lambda i: (0, i, 0)),
        pl.BlockSpec((BR, 1), lambda i: (i, 0)),
        pl.BlockSpec((1, D), lambda i: (0, 0)),
    ],
    out_specs=pl.BlockSpec((BR, D), lambda i: (i, 0)),
    out_shape=jax.ShapeDtypeStruct((N, D), jnp.float32),
)


def kernel(x, edge_index, W1, b1, W2, b2, W3, b3):
    src = edge_index[0].astype(jnp.int32)
    dst = edge_index[1].astype(jnp.int32)
    src_s = src.reshape(NS, NCH_S, CH)
    dst_s = dst.reshape(NS, NCH_S, CH)
    dst_h = dst.reshape(NW, NCH_H, CH)
    zeros1 = jnp.zeros((N,), jnp.float32)
    zeros2 = jnp.zeros((N, DH), jnp.float32)

    hist_p = _sc_hist(dst_h, zeros1)
    hist_p = hist_p.reshape(NC, N, 1)
    dinv, y = _tc_prep(hist_p, x, W1)

    p = _sc_scatter(y, src_s, dst_s, zeros2)
    y = _tc_mid(p, y, dinv, b1.reshape(1, D), W2)

    p = _sc_scatter(y, src_s, dst_s, zeros2)
    y = _tc_mid(p, y, dinv, b2.reshape(1, D), W3)

    p = _sc_scatter(y, src_s, dst_s, zeros2)
    return _tc_final(p, y, dinv, b3.reshape(1, D))
